# Initial kernel scaffold; baseline (speedup 1.0000x reference)
import jax
import jax.numpy as jnp
from jax.experimental import pallas as pl


def kernel(x, edge_index, edge_weight, W1, b1, W2, b2):
    raise NotImplementedError("write your pallas kernel here")


# speedup vs baseline: 24.5542x; 24.5542x over previous
import jax
import jax.numpy as jnp
from jax import lax
from jax.experimental import pallas as pl
from jax.experimental.pallas import tpu as pltpu
from jax.experimental.pallas import tpu_sc as plsc

N = 10000
NP = 10240
NT = 640
FP = 8
FC = 4
E = 320000
EP = 327680
ET = 20480
CH = 2048
NCH = ET // CH
ALPHA = 0.1
ITERS = 10
DF = 128
DH = 64


def _mlp_body(x_ref, w1_ref, b1_ref, w2_ref, b2_ref, o_ref):
  h = jnp.dot(x_ref[...], w1_ref[...], preferred_element_type=jnp.float32)
  h = jnp.maximum(h + b1_ref[...], 0.0)
  o = jnp.dot(h, w2_ref[...], preferred_element_type=jnp.float32)
  o_ref[...] = o + b2_ref[...]


def _mlp(x_p, W1, b1, W2p, b2p):
  BM = 1024
  return pl.pallas_call(
      _mlp_body,
      grid=(NP // BM,),
      in_specs=[
          pl.BlockSpec((BM, DF), lambda i: (i, 0)),
          pl.BlockSpec((DF, DH), lambda i: (0, 0)),
          pl.BlockSpec((1, DH), lambda i: (0, 0)),
          pl.BlockSpec((DH, FP), lambda i: (0, 0)),
          pl.BlockSpec((1, FP), lambda i: (0, 0)),
      ],
      out_specs=pl.BlockSpec((BM, FP), lambda i: (i, 0)),
      out_shape=jax.ShapeDtypeStruct((NP, FP), jnp.float32),
  )(x_p, W1, b1.reshape(1, DH), W2p, b2p.reshape(1, FP))


def _rsqrt_newton(x):
  y = 1.0 / x
  for _ in range(28):
    y = y * (1.5 - 0.5 * x * y * y)
  return y


def _appnp_body(h0_hbm, col_hbm, row_hbm, w_hbm, out_hbm,
                col_v, row_v, nw_v,
                gb0, gb1, gb2, gb3, hb0, hb1, hb2, hb3,
                ob0, ob1, ob2, ob3, ab0, ab1, ab2, ab3,
                dis_v, dis2_v,
                h_s0, h_s1, h_s2, h_s3, a_s0, a_s1, a_s2, a_s3, dis_s):
  gbuf = [gb0, gb1, gb2, gb3]
  hbuf = [hb0, hb1, hb2, hb3]
  h0buf = [ob0, ob1, ob2, ob3]
  abuf = [ab0, ab1, ab2, ab3]
  h_s = [h_s0, h_s1, h_s2, h_s3]
  a_s = [a_s0, a_s1, a_s2, a_s3]
  cid = lax.axis_index("c")
  sid = lax.axis_index("s")
  base_n = sid * NT

  iota16 = lax.iota(jnp.int32, 16)
  zeros16 = jnp.zeros((16,), jnp.float32)
  zeros16i = jnp.zeros((16,), jnp.int32)
  own = pl.ds(base_n, NT)

  pltpu.sync_copy(col_hbm.at[sid], col_v)
  pltpu.sync_copy(row_hbm.at[sid], row_v)
  pltpu.sync_copy(w_hbm.at[sid], nw_v)
  for f in range(FC):
    pltpu.sync_copy(h0_hbm.at[cid, f, own], h0buf[f])
    pltpu.sync_copy(h0_hbm.at[cid, f, own], hbuf[f])
    pltpu.sync_copy(h0buf[f], h_s[f].at[own])

  def azero(v, _):
    abuf[0][pl.ds(16 * v, 16)] = zeros16
    return 0
  lax.fori_loop(0, NT // 16, azero, 0)
  for f in range(FC):
    pltpu.sync_copy(abuf[0], a_s[f].at[own])
  plsc.subcore_barrier()

  for c in range(NCH):
    pltpu.sync_copy(nw_v.at[pl.ds(c * CH, CH)], a_s0.at[row_v.at[pl.ds(c * CH, CH)]],
                    add=True)
  plsc.subcore_barrier()

  pltpu.sync_copy(a_s0.at[own], abuf[0])

  def dis_calc(v, _):
    dv = abuf[0][pl.ds(16 * v, 16)] + 1.0
    d = _rsqrt_newton(dv)
    dis_v[pl.ds(16 * v, 16)] = d
    dis2_v[pl.ds(16 * v, 16)] = d * d
    abuf[0][pl.ds(16 * v, 16)] = zeros16
    return 0
  lax.fori_loop(0, NT // 16, dis_calc, 0)

  pltpu.sync_copy(dis_v, dis_s.at[own])
  pltpu.sync_copy(abuf[0], a_s0.at[own])
  plsc.subcore_barrier()

  for c in range(NCH):
    pltpu.sync_copy(dis_s.at[row_v.at[pl.ds(c * CH, CH)]], gbuf[0])
    pltpu.sync_copy(dis_s.at[col_v.at[pl.ds(c * CH, CH)]], gbuf[1])

    def nw_calc(v, _):
      base_e = pl.ds(c * CH + 16 * v, 16)
      ds16 = pl.ds(16 * v, 16)
      nw_v[base_e] = gbuf[0][ds16] * nw_v[base_e] * gbuf[1][ds16]
      return 0
    lax.fori_loop(0, CH // 16, nw_calc, 0)

  def one_iter(_, carry):
    for c in range(NCH):
      for f in range(FC):
        pltpu.sync_copy(h_s[f].at[col_v.at[pl.ds(c * CH, CH)]], gbuf[f])

      def scale(v, _):
        nwv = nw_v[pl.ds(c * CH + 16 * v, 16)]
        for f in range(FC):
          gbuf[f][pl.ds(16 * v, 16)] = gbuf[f][pl.ds(16 * v, 16)] * nwv
        return 0
      lax.fori_loop(0, CH // 16, scale, 0)

      for f in range(FC):
        pltpu.sync_copy(gbuf[f], a_s[f].at[row_v.at[pl.ds(c * CH, CH)]], add=True)

    plsc.subcore_barrier()

    for f in range(FC):
      pltpu.sync_copy(a_s[f].at[own], abuf[f])

    def update(v, _):
      ds16 = pl.ds(16 * v, 16)
      d2 = dis2_v[ds16]
      for f in range(FC):
        hn = ((1.0 - ALPHA) * (abuf[f][ds16] + d2 * hbuf[f][ds16])
              + ALPHA * h0buf[f][ds16])
        hbuf[f][ds16] = hn
        abuf[f][ds16] = jnp.zeros((16,), jnp.float32)
      return 0
    lax.fori_loop(0, NT // 16, update, 0)

    for f in range(FC):
      pltpu.sync_copy(hbuf[f], h_s[f].at[own])
      pltpu.sync_copy(abuf[f], a_s[f].at[own])
    plsc.subcore_barrier()
    return carry

  lax.fori_loop(0, ITERS, one_iter, 0)

  for f in range(FC):
    pltpu.sync_copy(hbuf[f], out_hbm.at[cid, f, own])


def _appnp(h0_t, col_p, row_p, w_p):
  mesh = plsc.VectorSubcoreMesh(core_axis_name="c", subcore_axis_name="s",
                                num_cores=2, num_subcores=16)
  f = pl.kernel(
      _appnp_body,
      out_type=jax.ShapeDtypeStruct((2, FC, NP), jnp.float32),
      mesh=mesh,
      scratch_types=[
          pltpu.VMEM((ET,), jnp.int32),
          pltpu.VMEM((ET,), jnp.int32),
          pltpu.VMEM((ET,), jnp.float32),
          pltpu.VMEM((CH,), jnp.float32),
          pltpu.VMEM((CH,), jnp.float32),
          pltpu.VMEM((CH,), jnp.float32),
          pltpu.VMEM((CH,), jnp.float32),
          pltpu.VMEM((NT,), jnp.float32),
          pltpu.VMEM((NT,), jnp.float32),
          pltpu.VMEM((NT,), jnp.float32),
          pltpu.VMEM((NT,), jnp.float32),
          pltpu.VMEM((NT,), jnp.float32),
          pltpu.VMEM((NT,), jnp.float32),
          pltpu.VMEM((NT,), jnp.float32),
          pltpu.VMEM((NT,), jnp.float32),
          pltpu.VMEM((NT,), jnp.float32),
          pltpu.VMEM((NT,), jnp.float32),
          pltpu.VMEM((NT,), jnp.float32),
          pltpu.VMEM((NT,), jnp.float32),
          pltpu.VMEM((NT,), jnp.float32),
          pltpu.VMEM((NT,), jnp.float32),
          pltpu.VMEM_SHARED((NP,), jnp.float32),
          pltpu.VMEM_SHARED((NP,), jnp.float32),
          pltpu.VMEM_SHARED((NP,), jnp.float32),
          pltpu.VMEM_SHARED((NP,), jnp.float32),
          pltpu.VMEM_SHARED((NP,), jnp.float32),
          pltpu.VMEM_SHARED((NP,), jnp.float32),
          pltpu.VMEM_SHARED((NP,), jnp.float32),
          pltpu.VMEM_SHARED((NP,), jnp.float32),
          pltpu.VMEM_SHARED((NP,), jnp.float32),
      ],
  )
  return f(h0_t, col_p, row_p, w_p)


@jax.jit
def kernel(x, edge_index, edge_weight, W1, b1, W2, b2):
  x_p = jnp.pad(x, ((0, NP - N), (0, 0)))
  W2p = jnp.pad(W2, ((0, 0), (0, FP - W2.shape[1])))
  b2p = jnp.pad(b2, (0, FP - b2.shape[0]))

  h0 = _mlp(x_p, W1, b1, W2p, b2p)
  h0_t = h0.T.reshape(2, FC, NP)

  npad = EP - E
  pad_idx = (jnp.arange(npad, dtype=jnp.int32) * 131) % N
  row_p = jnp.concatenate([edge_index[0], pad_idx]).reshape(16, ET)
  col_p = jnp.concatenate([edge_index[1], pad_idx]).reshape(16, ET)
  w_p = jnp.concatenate(
      [edge_weight, jnp.zeros((npad,), jnp.float32)]).reshape(16, ET)

  out = _appnp(h0_t, col_p, row_p, w_p)
  out = out.reshape(FP, NP).T
  return out[:N, :6]


# speedup vs baseline: 42.7109x; 1.7395x over previous
import jax
import jax.numpy as jnp
from jax import lax
from jax.experimental import pallas as pl
from jax.experimental.pallas import tpu as pltpu
from jax.experimental.pallas import tpu_sc as plsc

N = 10000
NP = 10240
NT = 640
FP = 8
FC = 4
E = 320000
EP = 327680
ET = 20480
CH = 2048
NCH = ET // CH
ALPHA = 0.1
ITERS = 10
DF = 128
DH = 64


def _mlp_body(x_ref, w1_ref, b1_ref, w2_ref, b2_ref, o_ref):
  h = jnp.dot(x_ref[...], w1_ref[...], preferred_element_type=jnp.float32)
  h = jnp.maximum(h + b1_ref[...], 0.0)
  o = jnp.dot(h, w2_ref[...], preferred_element_type=jnp.float32)
  o_ref[...] = o + b2_ref[...]


def _mlp(x_p, W1, b1, W2p, b2p):
  BM = 1024
  return pl.pallas_call(
      _mlp_body,
      grid=(NP // BM,),
      in_specs=[
          pl.BlockSpec((BM, DF), lambda i: (i, 0)),
          pl.BlockSpec((DF, DH), lambda i: (0, 0)),
          pl.BlockSpec((1, DH), lambda i: (0, 0)),
          pl.BlockSpec((DH, FP), lambda i: (0, 0)),
          pl.BlockSpec((1, FP), lambda i: (0, 0)),
      ],
      out_specs=pl.BlockSpec((BM, FP), lambda i: (i, 0)),
      out_shape=jax.ShapeDtypeStruct((NP, FP), jnp.float32),
  )(x_p, W1, b1.reshape(1, DH), W2p, b2p.reshape(1, FP))


def _rsqrt_newton(x):
  y = 1.0 / x
  for _ in range(28):
    y = y * (1.5 - 0.5 * x * y * y)
  return y


def _appnp_body(h0_hbm, col_hbm, row_hbm, w_hbm, out_hbm,
                col_v, row_v, nw_v,
                gb0, gb1, gb2, gb3, gb4, gb5, gb6, gb7,
                gb8, gb9, gb10, gb11, hb0, hb1, hb2, hb3,
                ob0, ob1, ob2, ob3, ab0, ab1, ab2, ab3,
                dis_v, dis2_v,
                gs0, gs1, gs2, ss0, ss1, ss2,
                h_s0, h_s1, h_s2, h_s3, a_s0, a_s1, a_s2, a_s3, dis_s):
  gset = [[gb0, gb1, gb2, gb3], [gb4, gb5, gb6, gb7], [gb8, gb9, gb10, gb11]]
  gbuf = gset[0]
  gsem = [gs0, gs1, gs2]
  ssem = [ss0, ss1, ss2]
  hbuf = [hb0, hb1, hb2, hb3]
  h0buf = [ob0, ob1, ob2, ob3]
  abuf = [ab0, ab1, ab2, ab3]
  h_s = [h_s0, h_s1, h_s2, h_s3]
  a_s = [a_s0, a_s1, a_s2, a_s3]
  cid = lax.axis_index("c")
  sid = lax.axis_index("s")
  base_n = sid * NT

  iota16 = lax.iota(jnp.int32, 16)
  zeros16 = jnp.zeros((16,), jnp.float32)
  zeros16i = jnp.zeros((16,), jnp.int32)
  own = pl.ds(base_n, NT)

  pltpu.sync_copy(col_hbm.at[sid], col_v)
  pltpu.sync_copy(row_hbm.at[sid], row_v)
  pltpu.sync_copy(w_hbm.at[sid], nw_v)
  for f in range(FC):
    pltpu.sync_copy(h0_hbm.at[cid, f, own], h0buf[f])
    pltpu.sync_copy(h0_hbm.at[cid, f, own], hbuf[f])
    pltpu.sync_copy(h0buf[f], h_s[f].at[own])

  def azero(v, _):
    abuf[0][pl.ds(16 * v, 16)] = zeros16
    return 0
  lax.fori_loop(0, NT // 16, azero, 0)
  for f in range(FC):
    pltpu.sync_copy(abuf[0], a_s[f].at[own])
  plsc.subcore_barrier()

  for c in range(NCH):
    pltpu.sync_copy(nw_v.at[pl.ds(c * CH, CH)], a_s0.at[row_v.at[pl.ds(c * CH, CH)]],
                    add=True)
  plsc.subcore_barrier()

  pltpu.sync_copy(a_s0.at[own], abuf[0])

  def dis_calc(v, _):
    dv = abuf[0][pl.ds(16 * v, 16)] + 1.0
    d = _rsqrt_newton(dv)
    dis_v[pl.ds(16 * v, 16)] = d
    dis2_v[pl.ds(16 * v, 16)] = d * d
    abuf[0][pl.ds(16 * v, 16)] = zeros16
    return 0
  lax.fori_loop(0, NT // 16, dis_calc, 0)

  pltpu.sync_copy(dis_v, dis_s.at[own])
  pltpu.sync_copy(abuf[0], a_s0.at[own])
  plsc.subcore_barrier()

  for c in range(NCH):
    pltpu.sync_copy(dis_s.at[row_v.at[pl.ds(c * CH, CH)]], gbuf[0])
    pltpu.sync_copy(dis_s.at[col_v.at[pl.ds(c * CH, CH)]], gbuf[1])

    def nw_calc(v, _):
      base_e = pl.ds(c * CH + 16 * v, 16)
      ds16 = pl.ds(16 * v, 16)
      nw_v[base_e] = gbuf[0][ds16] * nw_v[base_e] * gbuf[1][ds16]
      return 0
    lax.fori_loop(0, CH // 16, nw_calc, 0)

  def fire_gather(c, b):
    off = pl.ds(c * CH, CH)
    return [pltpu.async_copy(h_s[f].at[col_v.at[off]], gset[b][f], gsem[b])
            for f in range(FC)]

  def fire_scatter(c, b):
    off = pl.ds(c * CH, CH)
    return [pltpu.async_copy(gset[b][f], a_s[f].at[row_v.at[off]], ssem[b],
                             add=True)
            for f in range(FC)]

  def one_iter(_, carry):
    gd = {0: fire_gather(0, 0), 1: fire_gather(1, 1)}
    sd = {}
    for c in range(NCH):
      b = c % 3
      for d in gd.pop(c):
        d.wait()

      def scale(v, _):
        nwv = nw_v[pl.ds(c * CH + 16 * v, 16)]
        for f in range(FC):
          gset[b][f][pl.ds(16 * v, 16)] = gset[b][f][pl.ds(16 * v, 16)] * nwv
        return 0
      lax.fori_loop(0, CH // 16, scale, 0)

      if c - 1 >= 0:
        for d in sd.pop(c - 1):
          d.wait()
      sd[c] = fire_scatter(c, b)
      if c + 2 < NCH:
        gd[c + 2] = fire_gather(c + 2, (c + 2) % 3)
    for d in sd.pop(NCH - 1):
      d.wait()

    plsc.subcore_barrier()

    rd = [pltpu.async_copy(a_s[f].at[own], abuf[f], gsem[0])
          for f in range(FC)]
    for d in rd:
      d.wait()

    def update(v, _):
      ds16 = pl.ds(16 * v, 16)
      d2 = dis2_v[ds16]
      for f in range(FC):
        hn = ((1.0 - ALPHA) * (abuf[f][ds16] + d2 * hbuf[f][ds16])
              + ALPHA * h0buf[f][ds16])
        hbuf[f][ds16] = hn
        abuf[f][ds16] = jnp.zeros((16,), jnp.float32)
      return 0
    lax.fori_loop(0, NT // 16, update, 0)

    wd = [pltpu.async_copy(hbuf[f], h_s[f].at[own], ssem[0])
          for f in range(FC)]
    wd += [pltpu.async_copy(abuf[f], a_s[f].at[own], ssem[1])
           for f in range(FC)]
    for d in wd:
      d.wait()
    plsc.subcore_barrier()
    return carry

  lax.fori_loop(0, ITERS, one_iter, 0)

  for f in range(FC):
    pltpu.sync_copy(hbuf[f], out_hbm.at[cid, f, own])


def _appnp(h0_t, col_p, row_p, w_p):
  mesh = plsc.VectorSubcoreMesh(core_axis_name="c", subcore_axis_name="s",
                                num_cores=2, num_subcores=16)
  f = pl.kernel(
      _appnp_body,
      out_type=jax.ShapeDtypeStruct((2, FC, NP), jnp.float32),
      mesh=mesh,
      scratch_types=[
          pltpu.VMEM((ET,), jnp.int32),
          pltpu.VMEM((ET,), jnp.int32),
          pltpu.VMEM((ET,), jnp.float32),
          pltpu.VMEM((CH,), jnp.float32),
          pltpu.VMEM((CH,), jnp.float32),
          pltpu.VMEM((CH,), jnp.float32),
          pltpu.VMEM((CH,), jnp.float32),
          pltpu.VMEM((CH,), jnp.float32),
          pltpu.VMEM((CH,), jnp.float32),
          pltpu.VMEM((CH,), jnp.float32),
          pltpu.VMEM((CH,), jnp.float32),
          pltpu.VMEM((CH,), jnp.float32),
          pltpu.VMEM((CH,), jnp.float32),
          pltpu.VMEM((CH,), jnp.float32),
          pltpu.VMEM((CH,), jnp.float32),
          pltpu.VMEM((NT,), jnp.float32),
          pltpu.VMEM((NT,), jnp.float32),
          pltpu.VMEM((NT,), jnp.float32),
          pltpu.VMEM((NT,), jnp.float32),
          pltpu.VMEM((NT,), jnp.float32),
          pltpu.VMEM((NT,), jnp.float32),
          pltpu.VMEM((NT,), jnp.float32),
          pltpu.VMEM((NT,), jnp.float32),
          pltpu.VMEM((NT,), jnp.float32),
          pltpu.VMEM((NT,), jnp.float32),
          pltpu.VMEM((NT,), jnp.float32),
          pltpu.VMEM((NT,), jnp.float32),
          pltpu.VMEM((NT,), jnp.float32),
          pltpu.VMEM((NT,), jnp.float32),
          pltpu.SemaphoreType.DMA,
          pltpu.SemaphoreType.DMA,
          pltpu.SemaphoreType.DMA,
          pltpu.SemaphoreType.DMA,
          pltpu.SemaphoreType.DMA,
          pltpu.SemaphoreType.DMA,
          pltpu.VMEM_SHARED((NP,), jnp.float32),
          pltpu.VMEM_SHARED((NP,), jnp.float32),
          pltpu.VMEM_SHARED((NP,), jnp.float32),
          pltpu.VMEM_SHARED((NP,), jnp.float32),
          pltpu.VMEM_SHARED((NP,), jnp.float32),
          pltpu.VMEM_SHARED((NP,), jnp.float32),
          pltpu.VMEM_SHARED((NP,), jnp.float32),
          pltpu.VMEM_SHARED((NP,), jnp.float32),
          pltpu.VMEM_SHARED((NP,), jnp.float32),
      ],
  )
  return f(h0_t, col_p, row_p, w_p)


@jax.jit
def kernel(x, edge_index, edge_weight, W1, b1, W2, b2):
  x_p = jnp.pad(x, ((0, NP - N), (0, 0)))
  W2p = jnp.pad(W2, ((0, 0), (0, FP - W2.shape[1])))
  b2p = jnp.pad(b2, (0, FP - b2.shape[0]))

  h0 = _mlp(x_p, W1, b1, W2p, b2p)
  h0_t = h0.T.reshape(2, FC, NP)

  npad = EP - E
  pad_idx = (jnp.arange(npad, dtype=jnp.int32) * 131) % N
  row_p = jnp.concatenate([edge_index[0], pad_idx]).reshape(16, ET)
  col_p = jnp.concatenate([edge_index[1], pad_idx]).reshape(16, ET)
  w_p = jnp.concatenate(
      [edge_weight, jnp.zeros((npad,), jnp.float32)]).reshape(16, ET)

  out = _appnp(h0_t, col_p, row_p, w_p)
  out = out.reshape(FP, NP).T
  return out[:N, :6]


# speedup vs baseline: 42.7616x; 1.0012x over previous
import jax
import jax.numpy as jnp
from jax import lax
from jax.experimental import pallas as pl
from jax.experimental.pallas import tpu as pltpu
from jax.experimental.pallas import tpu_sc as plsc

N = 10000
NP = 10240
NT = 640
FP = 8
FC = 4
E = 320000
EP = 327680
ET = 20480
CH = 4096
NCH = ET // CH
ALPHA = 0.1
ITERS = 10
DF = 128
DH = 64


def _mlp_body(x_ref, w1_ref, b1_ref, w2_ref, b2_ref, o_ref):
  h = jnp.dot(x_ref[...], w1_ref[...], preferred_element_type=jnp.float32)
  h = jnp.maximum(h + b1_ref[...], 0.0)
  o = jnp.dot(h, w2_ref[...], preferred_element_type=jnp.float32)
  o_ref[...] = o + b2_ref[...]


def _mlp(x_p, W1, b1, W2p, b2p):
  BM = 1024
  return pl.pallas_call(
      _mlp_body,
      grid=(NP // BM,),
      in_specs=[
          pl.BlockSpec((BM, DF), lambda i: (i, 0)),
          pl.BlockSpec((DF, DH), lambda i: (0, 0)),
          pl.BlockSpec((1, DH), lambda i: (0, 0)),
          pl.BlockSpec((DH, FP), lambda i: (0, 0)),
          pl.BlockSpec((1, FP), lambda i: (0, 0)),
      ],
      out_specs=pl.BlockSpec((BM, FP), lambda i: (i, 0)),
      out_shape=jax.ShapeDtypeStruct((NP, FP), jnp.float32),
  )(x_p, W1, b1.reshape(1, DH), W2p, b2p.reshape(1, FP))


def _rsqrt_newton(x):
  y = 1.0 / x
  for _ in range(28):
    y = y * (1.5 - 0.5 * x * y * y)
  return y


def _appnp_body(h0_hbm, col_hbm, row_hbm, w_hbm, out_hbm,
                col_v, row_v, nw_v,
                gb0, gb1, gb2, gb3, gb4, gb5, gb6, gb7,
                gb8, gb9, gb10, gb11, hb0, hb1, hb2, hb3,
                ob0, ob1, ob2, ob3, ab0, ab1, ab2, ab3,
                dis_v, dis2_v,
                gs0, gs1, gs2, ss0, ss1, ss2,
                h_s0, h_s1, h_s2, h_s3, a_s0, a_s1, a_s2, a_s3, dis_s):
  gset = [[gb0, gb1, gb2, gb3], [gb4, gb5, gb6, gb7], [gb8, gb9, gb10, gb11]]
  gbuf = gset[0]
  gsem = [gs0, gs1, gs2]
  ssem = [ss0, ss1, ss2]
  hbuf = [hb0, hb1, hb2, hb3]
  h0buf = [ob0, ob1, ob2, ob3]
  abuf = [ab0, ab1, ab2, ab3]
  h_s = [h_s0, h_s1, h_s2, h_s3]
  a_s = [a_s0, a_s1, a_s2, a_s3]
  cid = lax.axis_index("c")
  sid = lax.axis_index("s")
  base_n = sid * NT

  iota16 = lax.iota(jnp.int32, 16)
  zeros16 = jnp.zeros((16,), jnp.float32)
  zeros16i = jnp.zeros((16,), jnp.int32)
  own = pl.ds(base_n, NT)

  pltpu.sync_copy(col_hbm.at[sid], col_v)
  pltpu.sync_copy(row_hbm.at[sid], row_v)
  pltpu.sync_copy(w_hbm.at[sid], nw_v)
  for f in range(FC):
    pltpu.sync_copy(h0_hbm.at[cid, f, own], h0buf[f])
    pltpu.sync_copy(h0_hbm.at[cid, f, own], hbuf[f])
    pltpu.sync_copy(h0buf[f], h_s[f].at[own])

  def azero(v, _):
    abuf[0][pl.ds(16 * v, 16)] = zeros16
    return 0
  lax.fori_loop(0, NT // 16, azero, 0)
  for f in range(FC):
    pltpu.sync_copy(abuf[0], a_s[f].at[own])
  plsc.subcore_barrier()

  for c in range(NCH):
    pltpu.sync_copy(nw_v.at[pl.ds(c * CH, CH)], a_s0.at[row_v.at[pl.ds(c * CH, CH)]],
                    add=True)
  plsc.subcore_barrier()

  pltpu.sync_copy(a_s0.at[own], abuf[0])

  def dis_calc(v, _):
    dv = abuf[0][pl.ds(16 * v, 16)] + 1.0
    d = _rsqrt_newton(dv)
    dis_v[pl.ds(16 * v, 16)] = d
    dis2_v[pl.ds(16 * v, 16)] = d * d
    abuf[0][pl.ds(16 * v, 16)] = zeros16
    return 0
  lax.fori_loop(0, NT // 16, dis_calc, 0)

  pltpu.sync_copy(dis_v, dis_s.at[own])
  pltpu.sync_copy(abuf[0], a_s0.at[own])
  plsc.subcore_barrier()

  for c in range(NCH):
    pltpu.sync_copy(dis_s.at[row_v.at[pl.ds(c * CH, CH)]], gbuf[0])
    pltpu.sync_copy(dis_s.at[col_v.at[pl.ds(c * CH, CH)]], gbuf[1])

    def nw_calc(v, _):
      base_e = pl.ds(c * CH + 16 * v, 16)
      ds16 = pl.ds(16 * v, 16)
      nw_v[base_e] = gbuf[0][ds16] * nw_v[base_e] * gbuf[1][ds16]
      return 0
    lax.fori_loop(0, CH // 16, nw_calc, 0)

  def fire_gather(c, b):
    off = pl.ds(c * CH, CH)
    return [pltpu.async_copy(h_s[f].at[col_v.at[off]], gset[b][f], gsem[b])
            for f in range(FC)]

  def fire_scatter(c, b):
    off = pl.ds(c * CH, CH)
    return [pltpu.async_copy(gset[b][f], a_s[f].at[row_v.at[off]], ssem[b],
                             add=True)
            for f in range(FC)]

  def one_iter(_, carry):
    gd = {0: fire_gather(0, 0), 1: fire_gather(1, 1)}
    sd = {}
    for c in range(NCH):
      b = c % 3
      for d in gd.pop(c):
        d.wait()

      def scale(v, _):
        nwv = nw_v[pl.ds(c * CH + 16 * v, 16)]
        for f in range(FC):
          gset[b][f][pl.ds(16 * v, 16)] = gset[b][f][pl.ds(16 * v, 16)] * nwv
        return 0
      lax.fori_loop(0, CH // 16, scale, 0)

      if c - 1 >= 0:
        for d in sd.pop(c - 1):
          d.wait()
      sd[c] = fire_scatter(c, b)
      if c + 2 < NCH:
        gd[c + 2] = fire_gather(c + 2, (c + 2) % 3)
    for d in sd.pop(NCH - 1):
      d.wait()

    plsc.subcore_barrier()

    rd = [pltpu.async_copy(a_s[f].at[own], abuf[f], gsem[0])
          for f in range(FC)]
    for d in rd:
      d.wait()

    def update(v, _):
      ds16 = pl.ds(16 * v, 16)
      d2 = dis2_v[ds16]
      for f in range(FC):
        hn = ((1.0 - ALPHA) * (abuf[f][ds16] + d2 * hbuf[f][ds16])
              + ALPHA * h0buf[f][ds16])
        hbuf[f][ds16] = hn
        abuf[f][ds16] = jnp.zeros((16,), jnp.float32)
      return 0
    lax.fori_loop(0, NT // 16, update, 0)

    wd = [pltpu.async_copy(hbuf[f], h_s[f].at[own], ssem[0])
          for f in range(FC)]
    wd += [pltpu.async_copy(abuf[f], a_s[f].at[own], ssem[1])
           for f in range(FC)]
    for d in wd:
      d.wait()
    plsc.subcore_barrier()
    return carry

  lax.fori_loop(0, ITERS, one_iter, 0)

  for f in range(FC):
    pltpu.sync_copy(hbuf[f], out_hbm.at[cid, f, own])


def _appnp(h0_t, col_p, row_p, w_p):
  mesh = plsc.VectorSubcoreMesh(core_axis_name="c", subcore_axis_name="s",
                                num_cores=2, num_subcores=16)
  f = pl.kernel(
      _appnp_body,
      out_type=jax.ShapeDtypeStruct((2, FC, NP), jnp.float32),
      mesh=mesh,
      scratch_types=[
          pltpu.VMEM((ET,), jnp.int32),
          pltpu.VMEM((ET,), jnp.int32),
          pltpu.VMEM((ET,), jnp.float32),
          pltpu.VMEM((CH,), jnp.float32),
          pltpu.VMEM((CH,), jnp.float32),
          pltpu.VMEM((CH,), jnp.float32),
          pltpu.VMEM((CH,), jnp.float32),
          pltpu.VMEM((CH,), jnp.float32),
          pltpu.VMEM((CH,), jnp.float32),
          pltpu.VMEM((CH,), jnp.float32),
          pltpu.VMEM((CH,), jnp.float32),
          pltpu.VMEM((CH,), jnp.float32),
          pltpu.VMEM((CH,), jnp.float32),
          pltpu.VMEM((CH,), jnp.float32),
          pltpu.VMEM((CH,), jnp.float32),
          pltpu.VMEM((NT,), jnp.float32),
          pltpu.VMEM((NT,), jnp.float32),
          pltpu.VMEM((NT,), jnp.float32),
          pltpu.VMEM((NT,), jnp.float32),
          pltpu.VMEM((NT,), jnp.float32),
          pltpu.VMEM((NT,), jnp.float32),
          pltpu.VMEM((NT,), jnp.float32),
          pltpu.VMEM((NT,), jnp.float32),
          pltpu.VMEM((NT,), jnp.float32),
          pltpu.VMEM((NT,), jnp.float32),
          pltpu.VMEM((NT,), jnp.float32),
          pltpu.VMEM((NT,), jnp.float32),
          pltpu.VMEM((NT,), jnp.float32),
          pltpu.VMEM((NT,), jnp.float32),
          pltpu.SemaphoreType.DMA,
          pltpu.SemaphoreType.DMA,
          pltpu.SemaphoreType.DMA,
          pltpu.SemaphoreType.DMA,
          pltpu.SemaphoreType.DMA,
          pltpu.SemaphoreType.DMA,
          pltpu.VMEM_SHARED((NP,), jnp.float32),
          pltpu.VMEM_SHARED((NP,), jnp.float32),
          pltpu.VMEM_SHARED((NP,), jnp.float32),
          pltpu.VMEM_SHARED((NP,), jnp.float32),
          pltpu.VMEM_SHARED((NP,), jnp.float32),
          pltpu.VMEM_SHARED((NP,), jnp.float32),
          pltpu.VMEM_SHARED((NP,), jnp.float32),
          pltpu.VMEM_SHARED((NP,), jnp.float32),
          pltpu.VMEM_SHARED((NP,), jnp.float32),
      ],
  )
  return f(h0_t, col_p, row_p, w_p)


@jax.jit
def kernel(x, edge_index, edge_weight, W1, b1, W2, b2):
  x_p = jnp.pad(x, ((0, NP - N), (0, 0)))
  W2p = jnp.pad(W2, ((0, 0), (0, FP - W2.shape[1])))
  b2p = jnp.pad(b2, (0, FP - b2.shape[0]))

  h0 = _mlp(x_p, W1, b1, W2p, b2p)
  h0_t = h0.T.reshape(2, FC, NP)

  npad = EP - E
  pad_idx = (jnp.arange(npad, dtype=jnp.int32) * 131) % N
  row_p = jnp.concatenate([edge_index[0], pad_idx]).reshape(16, ET)
  col_p = jnp.concatenate([edge_index[1], pad_idx]).reshape(16, ET)
  w_p = jnp.concatenate(
      [edge_weight, jnp.zeros((npad,), jnp.float32)]).reshape(16, ET)

  out = _appnp(h0_t, col_p, row_p, w_p)
  out = out.reshape(FP, NP).T
  return out[:N, :6]
